# baseline (speedup 1.0000x reference)
import jax
import jax.numpy as jnp
from jax import lax
from jax.experimental import pallas as pl

B = 8
NL = 13
L = 512
D = 768
NBINS = 320
DH = D // 2

_HI = lax.Precision.HIGHEST


def _body(vq_ref, xlo_ref, xhi_ref, o_ref):
    v = vq_ref[0]
    ixc = v[:, 0:1]
    iyc = v[:, 1:2]
    iota = lax.broadcasted_iota(jnp.int32, (L, NBINS), 1)
    ex = (ixc == iota).astype(jnp.float32)
    ey = (iyc == iota).astype(jnp.float32)
    cx = jnp.sum(ex, axis=0, keepdims=True)
    cy = jnp.sum(ey, axis=0, keepdims=True)
    fx = lax.dot_general(ex, cx, (((1,), (1,)), ((), ())), precision=_HI)
    fy = lax.dot_general(ey, cy, (((1,), (1,)), ((), ())), precision=_HI)
    r = 1.0 / (fx + fy)
    w = r * (1.0 / jnp.sum(r))
    olo = jnp.sum(xlo_ref[0, 0] * w, axis=0, keepdims=True)
    ohi = jnp.sum(xhi_ref[0, 0] * w, axis=0, keepdims=True)
    o_ref[0] = jnp.concatenate([olo, ohi], axis=1)


def kernel(input_feature, input_lengths, vq_indices):
    del input_lengths
    vq = vq_indices.astype(jnp.int32)
    out = pl.pallas_call(
        _body,
        grid=(B,),
        in_specs=[
            pl.BlockSpec((1, L, 2), lambda b: (b, 0, 0)),
            pl.BlockSpec((1, 1, L, DH), lambda b: (b, NL - 1, 0, 0)),
            pl.BlockSpec((1, 1, L, DH), lambda b: (b, NL - 1, 0, 1)),
        ],
        out_specs=pl.BlockSpec((1, 1, D), lambda b: (b, 0, 0)),
        out_shape=jax.ShapeDtypeStruct((B, 1, D), jnp.float32),
    )(vq, input_feature, input_feature)
    return out.reshape(B, D)
